# baseline (speedup 1.0000x reference)
import functools

import jax
import jax.numpy as jnp
from jax import lax
from jax.experimental import pallas as pl
from jax.experimental.pallas import tpu as pltpu
from jax.experimental.pallas import tpu_sc as plsc

NC = 2
NS = 16
NW = NC * NS
CHUNK = 128


def _node_pre(h_ref, wa_ref, wb_ref, b1_ref, ha_ref, hb_ref):
    h = h_ref[...]
    ha_ref[...] = jnp.dot(h, wa_ref[...], preferred_element_type=jnp.float32)
    hb_ref[...] = (jnp.dot(h, wb_ref[...], preferred_element_type=jnp.float32)
                   + b1_ref[...])


def _edge_mlp(ga_ref, gb_ref, d2_ref, w1c_ref, lng_ref, lnb_ref, w2_ref,
              b2_ref, m_ref):
    x = ga_ref[...] + gb_ref[...] + d2_ref[...] * w1c_ref[...]
    x = x * jax.nn.sigmoid(x)
    mu = jnp.mean(x, axis=-1, keepdims=True)
    var = jnp.mean((x - mu) ** 2, axis=-1, keepdims=True)
    x = (x - mu) * lax.rsqrt(var + 1e-5) * lng_ref[...] + lnb_ref[...]
    y = jnp.dot(x, w2_ref[...], preferred_element_type=jnp.float32) + b2_ref[...]
    m_ref[...] = y * jax.nn.sigmoid(y)


def _node_upd(h_ref, a0_ref, a1_ref, w3h_ref, w3a_ref, b3_ref, w4_ref, b4_ref,
              o_ref):
    h = h_ref[...]
    agg = a0_ref[...] + a1_ref[...]
    y = (jnp.dot(h, w3h_ref[...], preferred_element_type=jnp.float32)
         + jnp.dot(agg, w3a_ref[...], preferred_element_type=jnp.float32)
         + b3_ref[...])
    y = y * jax.nn.sigmoid(y)
    o_ref[...] = h + jnp.dot(y, w4_ref[...], preferred_element_type=jnp.float32) + b4_ref[...]


def _sc_gather(ha, hb, ei2, ej2, ga, gb, idxa, idxb,
               ba0, ba1, bb0, bb1, sg0, sg1, sw0, sw1):
    kb = idxa.shape[0]
    wid = lax.axis_index("c") * NS + lax.axis_index("s")
    pltpu.sync_copy(ei2.at[wid], idxa)
    pltpu.sync_copy(ej2.at[wid], idxb)
    bas, bbs, sgs, sws = (ba0, ba1), (bb0, bb1), (sg0, sg1), (sw0, sw1)

    def wait_gather(b):
        pltpu.make_async_copy(ha.at[pl.ds(0, CHUNK)], bas[b], sgs[b]).wait()
        pltpu.make_async_copy(hb.at[pl.ds(0, CHUNK)], bbs[b], sgs[b]).wait()

    def wait_write(b):
        pltpu.make_async_copy(ha.at[pl.ds(0, CHUNK)], bas[b], sws[b]).wait()
        pltpu.make_async_copy(hb.at[pl.ds(0, CHUNK)], bbs[b], sws[b]).wait()

    pltpu.async_copy(ha.at[idxa.at[0]], ba0, sg0)
    pltpu.async_copy(hb.at[idxb.at[0]], bb0, sg0)

    def step(b, other):
        def body(j, carry):
            @pl.when(j + 1 < kb)
            def _():
                @pl.when(j >= 1)
                def _():
                    wait_write(other)
                pltpu.async_copy(ha.at[idxa.at[j + 1]], bas[other], sgs[other])
                pltpu.async_copy(hb.at[idxb.at[j + 1]], bbs[other], sgs[other])
            wait_gather(b)
            row0 = (wid * kb + j) * CHUNK
            pltpu.async_copy(bas[b], ga.at[pl.ds(row0, CHUNK)], sws[b])
            pltpu.async_copy(bbs[b], gb.at[pl.ds(row0, CHUNK)], sws[b])
            return carry
        return body

    def pair(p, carry):
        carry = step(0, 1)(2 * p, carry)
        carry = step(1, 0)(2 * p + 1, carry)
        return carry

    carry = lax.fori_loop(0, kb // 2, pair, 0)
    if kb % 2:
        step(0, 1)(kb - 1, carry)
    wait_write(0)
    if kb > 1:
        wait_write(1)


def _sc_scatter(m, eis2, zrows, out, idx, b0, b1, sr0, sr1, sa0, sa1, shared):
    kb = idx.shape[0]
    np_rows = shared.shape[0]
    rpt = np_rows // NS
    cid = lax.axis_index("c")
    sid = lax.axis_index("s")
    w2 = cid * NS + sid
    bufs, srs, sas = (b0, b1), (sr0, sr1), (sa0, sa1)

    pltpu.sync_copy(eis2.at[w2], idx)
    pltpu.async_copy(m.at[pl.ds(w2 * kb * CHUNK, CHUNK)], b0, sr0)

    pltpu.sync_copy(zrows.at[pl.ds(sid * rpt, rpt)],
                    shared.at[pl.ds(sid * rpt, rpt)])
    plsc.subcore_barrier()

    def wait_read(b):
        pltpu.make_async_copy(m.at[pl.ds(0, CHUNK)], bufs[b], srs[b]).wait()

    def wait_add(b):
        pltpu.make_async_copy(m.at[pl.ds(0, CHUNK)], bufs[b], sas[b]).wait()

    def step(b, other):
        def body(j, carry):
            @pl.when(j + 1 < kb)
            def _():
                @pl.when(j >= 1)
                def _():
                    wait_add(other)
                row1 = (w2 * kb + j + 1) * CHUNK
                pltpu.async_copy(m.at[pl.ds(row1, CHUNK)], bufs[other],
                                 srs[other])
            wait_read(b)
            pltpu.async_copy(bufs[b], shared.at[idx.at[j]], sas[b], add=True)
            return carry
        return body

    def pair(p, carry):
        carry = step(0, 1)(2 * p, carry)
        carry = step(1, 0)(2 * p + 1, carry)
        return carry

    carry = lax.fori_loop(0, kb // 2, pair, 0)
    if kb % 2:
        step(0, 1)(kb - 1, carry)
    wait_add(0)
    if kb > 1:
        wait_add(1)
    plsc.subcore_barrier()

    pltpu.sync_copy(shared.at[pl.ds(sid * rpt, rpt)],
                    out.at[pl.ds(cid * np_rows + sid * rpt, rpt)])


def _ceil_to(x, m):
    return (x + m - 1) // m * m


def kernel(h, e, d2, W1, b1, ln_g, ln_b, W2, b2, W3, b3, W4, b4):
    n, d = h.shape
    num_e = e.shape[1]
    ei = e[0]
    ej = e[1]

    kb = -(-num_e // (NW * CHUNK))
    ep = NW * kb * CHUNK
    pad = ep - num_e
    np_rows = _ceil_to(n + 1, NS * 8)

    f32 = jnp.float32
    i32 = jnp.int32

    bn = 2000
    grid0 = n // bn
    hA, hB = pl.pallas_call(
        _node_pre,
        grid=(grid0,),
        in_specs=[
            pl.BlockSpec((bn, d), lambda i: (i, 0)),
            pl.BlockSpec((d, d), lambda i: (0, 0)),
            pl.BlockSpec((d, d), lambda i: (0, 0)),
            pl.BlockSpec((1, d), lambda i: (0, 0)),
        ],
        out_specs=[pl.BlockSpec((bn, d), lambda i: (i, 0)),
                   pl.BlockSpec((bn, d), lambda i: (i, 0))],
        out_shape=[jax.ShapeDtypeStruct((n, d), f32)] * 2,
        compiler_params=pltpu.CompilerParams(
            dimension_semantics=("parallel",)),
    )(h, W1[:d], W1[d:2 * d], b1.reshape(1, d))

    ei2 = jnp.concatenate([ei, jnp.zeros((pad,), i32)]).reshape(NW, kb, CHUNK)
    ej2 = jnp.concatenate([ej, jnp.zeros((pad,), i32)]).reshape(NW, kb, CHUNK)
    mesh = plsc.VectorSubcoreMesh(core_axis_name="c", subcore_axis_name="s",
                                  num_cores=NC, num_subcores=NS)
    gA, gB = pl.kernel(
        _sc_gather,
        out_type=[jax.ShapeDtypeStruct((ep, d), f32)] * 2,
        mesh=mesh,
        scratch_types=[
            pltpu.VMEM((kb, CHUNK), i32),
            pltpu.VMEM((kb, CHUNK), i32),
            pltpu.VMEM((CHUNK, d), f32),
            pltpu.VMEM((CHUNK, d), f32),
            pltpu.VMEM((CHUNK, d), f32),
            pltpu.VMEM((CHUNK, d), f32),
            pltpu.SemaphoreType.DMA,
            pltpu.SemaphoreType.DMA,
            pltpu.SemaphoreType.DMA,
            pltpu.SemaphoreType.DMA,
        ],
    )(hA, hB, ei2, ej2)

    be = 2048
    grid2 = ep // be
    d2p = jnp.concatenate([d2, jnp.zeros((pad, 1), f32)])
    m = pl.pallas_call(
        _edge_mlp,
        grid=(grid2,),
        in_specs=[
            pl.BlockSpec((be, d), lambda i: (i, 0)),
            pl.BlockSpec((be, d), lambda i: (i, 0)),
            pl.BlockSpec((be, 1), lambda i: (i, 0)),
            pl.BlockSpec((1, d), lambda i: (0, 0)),
            pl.BlockSpec((1, d), lambda i: (0, 0)),
            pl.BlockSpec((1, d), lambda i: (0, 0)),
            pl.BlockSpec((d, d), lambda i: (0, 0)),
            pl.BlockSpec((1, d), lambda i: (0, 0)),
        ],
        out_specs=pl.BlockSpec((be, d), lambda i: (i, 0)),
        out_shape=jax.ShapeDtypeStruct((ep, d), f32),
        compiler_params=pltpu.CompilerParams(
            dimension_semantics=("parallel",)),
    )(gA, gB, d2p, W1[2 * d].reshape(1, d), ln_g.reshape(1, d),
      ln_b.reshape(1, d), W2, b2.reshape(1, d))

    ei_s = jnp.concatenate([ei, jnp.full((pad,), n, i32)]).reshape(NW, kb, CHUNK)
    zrows = jnp.zeros((np_rows, d), f32)
    agg2 = pl.kernel(
        _sc_scatter,
        out_type=jax.ShapeDtypeStruct((NC * np_rows, d), f32),
        mesh=mesh,
        scratch_types=[
            pltpu.VMEM((kb, CHUNK), i32),
            pltpu.VMEM((CHUNK, d), f32),
            pltpu.VMEM((CHUNK, d), f32),
            pltpu.SemaphoreType.DMA,
            pltpu.SemaphoreType.DMA,
            pltpu.SemaphoreType.DMA,
            pltpu.SemaphoreType.DMA,
            pltpu.VMEM_SHARED((np_rows, d), f32),
        ],
    )(m, ei_s, zrows)

    out = pl.pallas_call(
        _node_upd,
        grid=(grid0,),
        in_specs=[
            pl.BlockSpec((bn, d), lambda i: (i, 0)),
            pl.BlockSpec((bn, d), lambda i: (i, 0)),
            pl.BlockSpec((bn, d), lambda i: (i, 0)),
            pl.BlockSpec((d, d), lambda i: (0, 0)),
            pl.BlockSpec((d, d), lambda i: (0, 0)),
            pl.BlockSpec((1, d), lambda i: (0, 0)),
            pl.BlockSpec((d, d), lambda i: (0, 0)),
            pl.BlockSpec((1, d), lambda i: (0, 0)),
        ],
        out_specs=pl.BlockSpec((bn, d), lambda i: (i, 0)),
        out_shape=jax.ShapeDtypeStruct((n, d), f32),
        compiler_params=pltpu.CompilerParams(
            dimension_semantics=("parallel",)),
    )(h, agg2[:n], agg2[np_rows:np_rows + n], W3[:d], W3[d:],
      b3.reshape(1, d), W4, b4.reshape(1, d))
    return out


# speedup vs baseline: 1.0746x; 1.0746x over previous
import functools

import jax
import jax.numpy as jnp
from jax import lax
from jax.experimental import pallas as pl
from jax.experimental.pallas import tpu as pltpu
from jax.experimental.pallas import tpu_sc as plsc

NC = 2
NS = 16
NW = NC * NS
CHUNK = 128


def _node_pre(h_ref, wa_ref, wb_ref, b1_ref, ha_ref, hb_ref):
    h = h_ref[...]
    ha_ref[...] = jnp.dot(h, wa_ref[...], preferred_element_type=jnp.float32)
    hb_ref[...] = (jnp.dot(h, wb_ref[...], preferred_element_type=jnp.float32)
                   + b1_ref[...])


def _edge_mlp(g_ref, d2_ref, w1c_ref, lng_ref, lnb_ref, w2_ref,
              b2_ref, m_ref):
    x = g_ref[...] + d2_ref[...] * w1c_ref[...]
    x = x * jax.nn.sigmoid(x)
    mu = jnp.mean(x, axis=-1, keepdims=True)
    var = jnp.mean((x - mu) ** 2, axis=-1, keepdims=True)
    x = (x - mu) * lax.rsqrt(var + 1e-5) * lng_ref[...] + lnb_ref[...]
    y = jnp.dot(x, w2_ref[...], preferred_element_type=jnp.float32) + b2_ref[...]
    m_ref[...] = y * jax.nn.sigmoid(y)


def _node_upd(h_ref, a0_ref, a1_ref, w3h_ref, w3a_ref, b3_ref, w4_ref, b4_ref,
              o_ref):
    h = h_ref[...]
    agg = a0_ref[...] + a1_ref[...]
    y = (jnp.dot(h, w3h_ref[...], preferred_element_type=jnp.float32)
         + jnp.dot(agg, w3a_ref[...], preferred_element_type=jnp.float32)
         + b3_ref[...])
    y = y * jax.nn.sigmoid(y)
    o_ref[...] = h + jnp.dot(y, w4_ref[...], preferred_element_type=jnp.float32) + b4_ref[...]


def _sc_gather(ha, hb, ei2, ej2, ga, idxa, idxb,
               ba0, ba1, sa0, sa1, sb0, sb1, sw0, sw1):
    kb = idxa.shape[0]
    wid = lax.axis_index("c") * NS + lax.axis_index("s")
    pltpu.sync_copy(ei2.at[wid], idxa)
    pltpu.sync_copy(ej2.at[wid], idxb)
    bas, sas, sbs, sws = (ba0, ba1), (sa0, sa1), (sb0, sb1), (sw0, sw1)

    def wait(b, sems):
        pltpu.make_async_copy(ha.at[pl.ds(0, CHUNK)], bas[b], sems[b]).wait()

    def step(b, other):
        def body(j, carry):
            @pl.when(j + 1 < kb)
            def _():
                @pl.when(j >= 1)
                def _():
                    wait(other, sws)
                pltpu.async_copy(ha.at[idxa.at[j + 1]], bas[other], sas[other])
            wait(b, sas)
            pltpu.async_copy(hb.at[idxb.at[j]], bas[b], sbs[b], add=True)
            wait(b, sbs)
            row0 = (wid * kb + j) * CHUNK
            pltpu.async_copy(bas[b], ga.at[pl.ds(row0, CHUNK)], sws[b])
            return carry
        return body

    pltpu.async_copy(ha.at[idxa.at[0]], ba0, sa0)

    def pair(p, carry):
        carry = step(0, 1)(2 * p, carry)
        carry = step(1, 0)(2 * p + 1, carry)
        return carry

    carry = lax.fori_loop(0, kb // 2, pair, 0)
    if kb % 2:
        step(0, 1)(kb - 1, carry)
    wait(0, sws)
    if kb > 1:
        wait(1, sws)


def _sc_scatter(m, eis2, zrows, out, idx, b0, b1, sr0, sr1, sa0, sa1, shared):
    kb = idx.shape[0]
    np_rows = shared.shape[0]
    rpt = np_rows // NS
    cid = lax.axis_index("c")
    sid = lax.axis_index("s")
    w2 = cid * NS + sid
    bufs, srs, sas = (b0, b1), (sr0, sr1), (sa0, sa1)

    pltpu.sync_copy(eis2.at[w2], idx)
    pltpu.async_copy(m.at[pl.ds(w2 * kb * CHUNK, CHUNK)], b0, sr0)

    pltpu.sync_copy(zrows.at[pl.ds(sid * rpt, rpt)],
                    shared.at[pl.ds(sid * rpt, rpt)])
    plsc.subcore_barrier()

    def wait_read(b):
        pltpu.make_async_copy(m.at[pl.ds(0, CHUNK)], bufs[b], srs[b]).wait()

    def wait_add(b):
        pltpu.make_async_copy(m.at[pl.ds(0, CHUNK)], bufs[b], sas[b]).wait()

    def step(b, other):
        def body(j, carry):
            @pl.when(j + 1 < kb)
            def _():
                @pl.when(j >= 1)
                def _():
                    wait_add(other)
                row1 = (w2 * kb + j + 1) * CHUNK
                pltpu.async_copy(m.at[pl.ds(row1, CHUNK)], bufs[other],
                                 srs[other])
            wait_read(b)
            pltpu.async_copy(bufs[b], shared.at[idx.at[j]], sas[b], add=True)
            return carry
        return body

    def pair(p, carry):
        carry = step(0, 1)(2 * p, carry)
        carry = step(1, 0)(2 * p + 1, carry)
        return carry

    carry = lax.fori_loop(0, kb // 2, pair, 0)
    if kb % 2:
        step(0, 1)(kb - 1, carry)
    wait_add(0)
    if kb > 1:
        wait_add(1)
    plsc.subcore_barrier()

    pltpu.sync_copy(shared.at[pl.ds(sid * rpt, rpt)],
                    out.at[pl.ds(cid * np_rows + sid * rpt, rpt)])


def _ceil_to(x, m):
    return (x + m - 1) // m * m


def kernel(h, e, d2, W1, b1, ln_g, ln_b, W2, b2, W3, b3, W4, b4):
    n, d = h.shape
    num_e = e.shape[1]
    ei = e[0]
    ej = e[1]

    kb = -(-num_e // (NW * CHUNK))
    ep = NW * kb * CHUNK
    pad = ep - num_e
    np_rows = _ceil_to(n + 1, NS * 8)

    f32 = jnp.float32
    i32 = jnp.int32

    bn = 2000
    grid0 = n // bn
    hA, hB = pl.pallas_call(
        _node_pre,
        grid=(grid0,),
        in_specs=[
            pl.BlockSpec((bn, d), lambda i: (i, 0)),
            pl.BlockSpec((d, d), lambda i: (0, 0)),
            pl.BlockSpec((d, d), lambda i: (0, 0)),
            pl.BlockSpec((1, d), lambda i: (0, 0)),
        ],
        out_specs=[pl.BlockSpec((bn, d), lambda i: (i, 0)),
                   pl.BlockSpec((bn, d), lambda i: (i, 0))],
        out_shape=[jax.ShapeDtypeStruct((n, d), f32)] * 2,
        compiler_params=pltpu.CompilerParams(
            dimension_semantics=("parallel",)),
    )(h, W1[:d], W1[d:2 * d], b1.reshape(1, d))

    ei2 = jnp.concatenate([ei, jnp.zeros((pad,), i32)]).reshape(NW, kb, CHUNK)
    ej2 = jnp.concatenate([ej, jnp.zeros((pad,), i32)]).reshape(NW, kb, CHUNK)
    mesh = plsc.VectorSubcoreMesh(core_axis_name="c", subcore_axis_name="s",
                                  num_cores=NC, num_subcores=NS)
    gsum = pl.kernel(
        _sc_gather,
        out_type=jax.ShapeDtypeStruct((ep, d), f32),
        mesh=mesh,
        scratch_types=[
            pltpu.VMEM((kb, CHUNK), i32),
            pltpu.VMEM((kb, CHUNK), i32),
            pltpu.VMEM((CHUNK, d), f32),
            pltpu.VMEM((CHUNK, d), f32),
            pltpu.SemaphoreType.DMA,
            pltpu.SemaphoreType.DMA,
            pltpu.SemaphoreType.DMA,
            pltpu.SemaphoreType.DMA,
            pltpu.SemaphoreType.DMA,
            pltpu.SemaphoreType.DMA,
        ],
    )(hA, hB, ei2, ej2)

    be = 2048
    grid2 = ep // be
    d2p = jnp.concatenate([d2, jnp.zeros((pad, 1), f32)])
    m = pl.pallas_call(
        _edge_mlp,
        grid=(grid2,),
        in_specs=[
            pl.BlockSpec((be, d), lambda i: (i, 0)),
            pl.BlockSpec((be, 1), lambda i: (i, 0)),
            pl.BlockSpec((1, d), lambda i: (0, 0)),
            pl.BlockSpec((1, d), lambda i: (0, 0)),
            pl.BlockSpec((1, d), lambda i: (0, 0)),
            pl.BlockSpec((d, d), lambda i: (0, 0)),
            pl.BlockSpec((1, d), lambda i: (0, 0)),
        ],
        out_specs=pl.BlockSpec((be, d), lambda i: (i, 0)),
        out_shape=jax.ShapeDtypeStruct((ep, d), f32),
        compiler_params=pltpu.CompilerParams(
            dimension_semantics=("parallel",)),
    )(gsum, d2p, W1[2 * d].reshape(1, d), ln_g.reshape(1, d),
      ln_b.reshape(1, d), W2, b2.reshape(1, d))

    ei_s = jnp.concatenate([ei, jnp.full((pad,), n, i32)]).reshape(NW, kb, CHUNK)
    zrows = jnp.zeros((np_rows, d), f32)
    agg2 = pl.kernel(
        _sc_scatter,
        out_type=jax.ShapeDtypeStruct((NC * np_rows, d), f32),
        mesh=mesh,
        scratch_types=[
            pltpu.VMEM((kb, CHUNK), i32),
            pltpu.VMEM((CHUNK, d), f32),
            pltpu.VMEM((CHUNK, d), f32),
            pltpu.SemaphoreType.DMA,
            pltpu.SemaphoreType.DMA,
            pltpu.SemaphoreType.DMA,
            pltpu.SemaphoreType.DMA,
            pltpu.VMEM_SHARED((np_rows, d), f32),
        ],
    )(m, ei_s, zrows)

    out = pl.pallas_call(
        _node_upd,
        grid=(grid0,),
        in_specs=[
            pl.BlockSpec((bn, d), lambda i: (i, 0)),
            pl.BlockSpec((bn, d), lambda i: (i, 0)),
            pl.BlockSpec((bn, d), lambda i: (i, 0)),
            pl.BlockSpec((d, d), lambda i: (0, 0)),
            pl.BlockSpec((d, d), lambda i: (0, 0)),
            pl.BlockSpec((1, d), lambda i: (0, 0)),
            pl.BlockSpec((d, d), lambda i: (0, 0)),
            pl.BlockSpec((1, d), lambda i: (0, 0)),
        ],
        out_specs=pl.BlockSpec((bn, d), lambda i: (i, 0)),
        out_shape=jax.ShapeDtypeStruct((n, d), f32),
        compiler_params=pltpu.CompilerParams(
            dimension_semantics=("parallel",)),
    )(h, agg2[:n], agg2[np_rows:np_rows + n], W3[:d], W3[d:],
      b3.reshape(1, d), W4, b4.reshape(1, d))
    return out


# speedup vs baseline: 1.3986x; 1.3015x over previous
import functools

import jax
import jax.numpy as jnp
from jax import lax
from jax.experimental import pallas as pl
from jax.experimental.pallas import tpu as pltpu
from jax.experimental.pallas import tpu_sc as plsc

NC = 2
NS = 16
NW = NC * NS
CHUNK = 128


def _node_pre(h_ref, wa_ref, wb_ref, b1_ref, ha_ref, hb_ref):
    h = h_ref[...]
    ha_ref[...] = jnp.dot(h, wa_ref[...], preferred_element_type=jnp.float32)
    hb_ref[...] = (jnp.dot(h, wb_ref[...], preferred_element_type=jnp.float32)
                   + b1_ref[...])


def _edge_mlp(ga_ref, gb_ref, d2_ref, w1c_ref, lng_ref, lnb_ref, w2_ref,
              b2_ref, m_ref):
    x = ga_ref[...] + gb_ref[...] + d2_ref[...] * w1c_ref[...]
    x = x * jax.nn.sigmoid(x)
    mu = jnp.mean(x, axis=-1, keepdims=True)
    var = jnp.mean((x - mu) ** 2, axis=-1, keepdims=True)
    x = (x - mu) * lax.rsqrt(var + 1e-5) * lng_ref[...] + lnb_ref[...]
    y = jnp.dot(x, w2_ref[...], preferred_element_type=jnp.float32) + b2_ref[...]
    m_ref[...] = y * jax.nn.sigmoid(y)


def _node_upd(h_ref, a0_ref, a1_ref, w3h_ref, w3a_ref, b3_ref, w4_ref, b4_ref,
              o_ref):
    h = h_ref[...]
    agg = a0_ref[...] + a1_ref[...]
    y = (jnp.dot(h, w3h_ref[...], preferred_element_type=jnp.float32)
         + jnp.dot(agg, w3a_ref[...], preferred_element_type=jnp.float32)
         + b3_ref[...])
    y = y * jax.nn.sigmoid(y)
    o_ref[...] = h + jnp.dot(y, w4_ref[...], preferred_element_type=jnp.float32) + b4_ref[...]


def _sc_gather(ha, hb, ei4, ej4, ga, gb, idx, b0v, b1v, sg0, sg1, sw0, sw1,
               tab):
    kbp = idx.shape[0]
    n2 = tab.shape[0]
    rpt = n2 // NS
    cid = lax.axis_index("c")
    sid = lax.axis_index("s")

    @pl.when(cid == 0)
    def _():
        pltpu.sync_copy(ha.at[pl.ds(sid * rpt, rpt)],
                        tab.at[pl.ds(sid * rpt, rpt)])

    @pl.when(cid == 1)
    def _():
        pltpu.sync_copy(hb.at[pl.ds(sid * rpt, rpt)],
                        tab.at[pl.ds(sid * rpt, rpt)])

    plsc.subcore_barrier()

    bufs, sgs, sws = (b0v, b1v), (sg0, sg1), (sw0, sw1)

    def wait(b, sems):
        pltpu.make_async_copy(ha.at[pl.ds(0, CHUNK)], bufs[b], sems[b]).wait()

    def phase(ph):
        @pl.when(cid == 0)
        def _():
            pltpu.sync_copy(ei4.at[sid, ph], idx)

        @pl.when(cid == 1)
        def _():
            pltpu.sync_copy(ej4.at[sid, ph], idx)

        pltpu.async_copy(tab.at[idx.at[0]], bufs[0], sgs[0])

        def step(b, other):
            def body(j, carry):
                @pl.when(j + 1 < kbp)
                def _():
                    @pl.when(j >= 1)
                    def _():
                        wait(other, sws)
                    pltpu.async_copy(tab.at[idx.at[j + 1]], bufs[other],
                                     sgs[other])
                wait(b, sgs)
                row0 = ((sid * 2 + ph) * kbp + j) * CHUNK

                @pl.when(cid == 0)
                def _():
                    pltpu.async_copy(bufs[b], ga.at[pl.ds(row0, CHUNK)],
                                     sws[b])

                @pl.when(cid == 1)
                def _():
                    pltpu.async_copy(bufs[b], gb.at[pl.ds(row0, CHUNK)],
                                     sws[b])
                return carry
            return body

        def pairfn(p, carry):
            carry = step(0, 1)(2 * p, carry)
            carry = step(1, 0)(2 * p + 1, carry)
            return carry

        carry = lax.fori_loop(0, kbp // 2, pairfn, 0)
        if kbp % 2:
            step(0, 1)(kbp - 1, carry)
        wait(0, sws)
        if kbp > 1:
            wait(1, sws)

    for ph in range(2):
        phase(ph)


def _sc_scatter(m, eis2, zrows, out, idx, b0, b1, sr0, sr1, sa0, sa1, shared):
    kb = idx.shape[0]
    np_rows = shared.shape[0]
    rpt = np_rows // NS
    cid = lax.axis_index("c")
    sid = lax.axis_index("s")
    w2 = cid * NS + sid
    bufs, srs, sas = (b0, b1), (sr0, sr1), (sa0, sa1)

    pltpu.sync_copy(eis2.at[w2], idx)
    pltpu.async_copy(m.at[pl.ds(w2 * kb * CHUNK, CHUNK)], b0, sr0)

    pltpu.sync_copy(zrows.at[pl.ds(sid * rpt, rpt)],
                    shared.at[pl.ds(sid * rpt, rpt)])
    plsc.subcore_barrier()

    def wait_read(b):
        pltpu.make_async_copy(m.at[pl.ds(0, CHUNK)], bufs[b], srs[b]).wait()

    def wait_add(b):
        pltpu.make_async_copy(m.at[pl.ds(0, CHUNK)], bufs[b], sas[b]).wait()

    def step(b, other):
        def body(j, carry):
            @pl.when(j + 1 < kb)
            def _():
                @pl.when(j >= 1)
                def _():
                    wait_add(other)
                row1 = (w2 * kb + j + 1) * CHUNK
                pltpu.async_copy(m.at[pl.ds(row1, CHUNK)], bufs[other],
                                 srs[other])
            wait_read(b)
            pltpu.async_copy(bufs[b], shared.at[idx.at[j]], sas[b], add=True)
            return carry
        return body

    def pair(p, carry):
        carry = step(0, 1)(2 * p, carry)
        carry = step(1, 0)(2 * p + 1, carry)
        return carry

    carry = lax.fori_loop(0, kb // 2, pair, 0)
    if kb % 2:
        step(0, 1)(kb - 1, carry)
    wait_add(0)
    if kb > 1:
        wait_add(1)
    plsc.subcore_barrier()

    pltpu.sync_copy(shared.at[pl.ds(sid * rpt, rpt)],
                    out.at[pl.ds(cid * np_rows + sid * rpt, rpt)])


def _ceil_to(x, m):
    return (x + m - 1) // m * m


def kernel(h, e, d2, W1, b1, ln_g, ln_b, W2, b2, W3, b3, W4, b4):
    n, d = h.shape
    num_e = e.shape[1]
    ei = e[0]
    ej = e[1]

    kb = -(-num_e // (NW * CHUNK))
    ep = NW * kb * CHUNK
    pad = ep - num_e
    f32 = jnp.float32
    i32 = jnp.int32

    np_rows = _ceil_to(n + 1, NS * 8)
    h_p = jnp.pad(h, ((0, np_rows - n), (0, 0)))
    bn0 = np_rows // 16
    hA, hB = pl.pallas_call(
        _node_pre,
        grid=(16,),
        in_specs=[
            pl.BlockSpec((bn0, d), lambda i: (i, 0)),
            pl.BlockSpec((d, d), lambda i: (0, 0)),
            pl.BlockSpec((d, d), lambda i: (0, 0)),
            pl.BlockSpec((1, d), lambda i: (0, 0)),
        ],
        out_specs=[pl.BlockSpec((bn0, d), lambda i: (i, 0)),
                   pl.BlockSpec((bn0, d), lambda i: (i, 0))],
        out_shape=[jax.ShapeDtypeStruct((np_rows, d), f32)] * 2,
        compiler_params=pltpu.CompilerParams(
            dimension_semantics=("parallel",)),
    )(h_p, W1[:d], W1[d:2 * d], b1.reshape(1, d))

    ei4 = jnp.concatenate([ei, jnp.zeros((pad,), i32)]).reshape(NS, 2, kb,
                                                                CHUNK)
    ej4 = jnp.concatenate([ej, jnp.zeros((pad,), i32)]).reshape(NS, 2, kb,
                                                                CHUNK)
    mesh = plsc.VectorSubcoreMesh(core_axis_name="c", subcore_axis_name="s",
                                  num_cores=NC, num_subcores=NS)
    gA, gB = pl.kernel(
        _sc_gather,
        out_type=[jax.ShapeDtypeStruct((ep, d), f32)] * 2,
        mesh=mesh,
        scratch_types=[
            pltpu.VMEM((kb, CHUNK), i32),
            pltpu.VMEM((CHUNK, d), f32),
            pltpu.VMEM((CHUNK, d), f32),
            pltpu.SemaphoreType.DMA,
            pltpu.SemaphoreType.DMA,
            pltpu.SemaphoreType.DMA,
            pltpu.SemaphoreType.DMA,
            pltpu.VMEM_SHARED((np_rows, d), f32),
        ],
    )(hA, hB, ei4, ej4)

    be = 2048
    grid2 = ep // be
    d2p = jnp.concatenate([d2, jnp.zeros((pad, 1), f32)])
    m = pl.pallas_call(
        _edge_mlp,
        grid=(grid2,),
        in_specs=[
            pl.BlockSpec((be, d), lambda i: (i, 0)),
            pl.BlockSpec((be, d), lambda i: (i, 0)),
            pl.BlockSpec((be, 1), lambda i: (i, 0)),
            pl.BlockSpec((1, d), lambda i: (0, 0)),
            pl.BlockSpec((1, d), lambda i: (0, 0)),
            pl.BlockSpec((1, d), lambda i: (0, 0)),
            pl.BlockSpec((d, d), lambda i: (0, 0)),
            pl.BlockSpec((1, d), lambda i: (0, 0)),
        ],
        out_specs=pl.BlockSpec((be, d), lambda i: (i, 0)),
        out_shape=jax.ShapeDtypeStruct((ep, d), f32),
        compiler_params=pltpu.CompilerParams(
            dimension_semantics=("parallel",)),
    )(gA, gB, d2p, W1[2 * d].reshape(1, d), ln_g.reshape(1, d),
      ln_b.reshape(1, d), W2, b2.reshape(1, d))

    ei_s = jnp.concatenate([ei, jnp.full((pad,), n, i32)]).reshape(NW, kb, CHUNK)
    zrows = jnp.zeros((np_rows, d), f32)
    agg2 = pl.kernel(
        _sc_scatter,
        out_type=jax.ShapeDtypeStruct((NC * np_rows, d), f32),
        mesh=mesh,
        scratch_types=[
            pltpu.VMEM((kb, CHUNK), i32),
            pltpu.VMEM((CHUNK, d), f32),
            pltpu.VMEM((CHUNK, d), f32),
            pltpu.SemaphoreType.DMA,
            pltpu.SemaphoreType.DMA,
            pltpu.SemaphoreType.DMA,
            pltpu.SemaphoreType.DMA,
            pltpu.VMEM_SHARED((np_rows, d), f32),
        ],
    )(m, ei_s, zrows)

    bn = 2000
    out = pl.pallas_call(
        _node_upd,
        grid=(n // bn,),
        in_specs=[
            pl.BlockSpec((bn, d), lambda i: (i, 0)),
            pl.BlockSpec((bn, d), lambda i: (i, 0)),
            pl.BlockSpec((bn, d), lambda i: (i, 0)),
            pl.BlockSpec((d, d), lambda i: (0, 0)),
            pl.BlockSpec((d, d), lambda i: (0, 0)),
            pl.BlockSpec((1, d), lambda i: (0, 0)),
            pl.BlockSpec((d, d), lambda i: (0, 0)),
            pl.BlockSpec((1, d), lambda i: (0, 0)),
        ],
        out_specs=pl.BlockSpec((bn, d), lambda i: (i, 0)),
        out_shape=jax.ShapeDtypeStruct((n, d), f32),
        compiler_params=pltpu.CompilerParams(
            dimension_semantics=("parallel",)),
    )(h, agg2[:n], agg2[np_rows:np_rows + n], W3[:d], W3[d:],
      b3.reshape(1, d), W4, b4.reshape(1, d))
    return out


# speedup vs baseline: 1.6343x; 1.1685x over previous
import jax
import jax.numpy as jnp
from jax import lax
from jax.experimental import pallas as pl
from jax.experimental.pallas import tpu as pltpu
from jax.experimental.pallas import tpu_sc as plsc

NC = 2
NS = 16
NW = NC * NS
CHUNK = 128
S = 4


def _node_pre(h_ref, wa_ref, wb_ref, b1_ref, ha_ref, hb_ref):
    h = h_ref[...]
    ha_ref[...] = jnp.dot(h, wa_ref[...], preferred_element_type=jnp.float32)
    hb_ref[...] = (jnp.dot(h, wb_ref[...], preferred_element_type=jnp.float32)
                   + b1_ref[...])


def _edge_mlp(ga_ref, gb_ref, d2_ref, w1c_ref, lng_ref, lnb_ref, w2_ref,
              b2_ref, m_ref):
    x = ga_ref[...] + gb_ref[...] + d2_ref[...] * w1c_ref[...]
    x = x * jax.nn.sigmoid(x)
    mu = jnp.mean(x, axis=-1, keepdims=True)
    var = jnp.mean((x - mu) ** 2, axis=-1, keepdims=True)
    x = (x - mu) * lax.rsqrt(var + 1e-5) * lng_ref[...] + lnb_ref[...]
    y = jnp.dot(x, w2_ref[...], preferred_element_type=jnp.float32) + b2_ref[...]
    m_ref[...] = y * jax.nn.sigmoid(y)


def _node_upd(h_ref, a0_ref, a1_ref, w3h_ref, w3a_ref, b3_ref, w4_ref, b4_ref,
              o_ref):
    h = h_ref[...]
    agg = a0_ref[...] + a1_ref[...]
    y = (jnp.dot(h, w3h_ref[...], preferred_element_type=jnp.float32)
         + jnp.dot(agg, w3a_ref[...], preferred_element_type=jnp.float32)
         + b3_ref[...])
    y = y * jax.nn.sigmoid(y)
    o_ref[...] = h + jnp.dot(y, w4_ref[...], preferred_element_type=jnp.float32) + b4_ref[...]


def _sc_gather(ha, hb, ei3, ej3, ga, gb, idx, b0v, b1v, sg0, sg1, sw0, sw1,
               tab):
    ks = idx.shape[0]
    n2 = tab.shape[0]
    rpt = n2 // NS
    cid = lax.axis_index("c")
    sid = lax.axis_index("s")

    @pl.when(cid == 0)
    def _():
        pltpu.sync_copy(ha.at[pl.ds(sid * rpt, rpt)],
                        tab.at[pl.ds(sid * rpt, rpt)])
        pltpu.sync_copy(ei3.at[sid], idx)

    @pl.when(cid == 1)
    def _():
        pltpu.sync_copy(hb.at[pl.ds(sid * rpt, rpt)],
                        tab.at[pl.ds(sid * rpt, rpt)])
        pltpu.sync_copy(ej3.at[sid], idx)

    plsc.subcore_barrier()

    bufs, sgs, sws = (b0v, b1v), (sg0, sg1), (sw0, sw1)

    def wait(b, sems):
        pltpu.make_async_copy(ha.at[pl.ds(0, CHUNK)], bufs[b], sems[b]).wait()

    pltpu.async_copy(tab.at[idx.at[0]], bufs[0], sgs[0])

    def step(b, other):
        def body(j, carry):
            @pl.when(j + 1 < ks)
            def _():
                @pl.when(j >= 1)
                def _():
                    wait(other, sws)
                pltpu.async_copy(tab.at[idx.at[j + 1]], bufs[other],
                                 sgs[other])
            wait(b, sgs)
            row0 = (sid * ks + j) * CHUNK

            @pl.when(cid == 0)
            def _():
                pltpu.async_copy(bufs[b], ga.at[pl.ds(row0, CHUNK)], sws[b])

            @pl.when(cid == 1)
            def _():
                pltpu.async_copy(bufs[b], gb.at[pl.ds(row0, CHUNK)], sws[b])
            return carry
        return body

    def pairfn(p, carry):
        carry = step(0, 1)(2 * p, carry)
        carry = step(1, 0)(2 * p + 1, carry)
        return carry

    carry = lax.fori_loop(0, ks // 2, pairfn, 0)
    if ks % 2:
        step(0, 1)(ks - 1, carry)
    wait(0, sws)
    if ks > 1:
        wait(1, sws)


def _sc_scatter(m0, m1, m2, m3, eis, zrows, out, idx, b0, b1v,
                sr0, sr1, sa0, sa1, shared):
    ms = (m0, m1, m2, m3)
    ksl = idx.shape[0]
    np_rows = shared.shape[0]
    rpt = np_rows // NS
    cid = lax.axis_index("c")
    sid = lax.axis_index("s")
    w2 = cid * NS + sid
    bufs, srs, sas = (b0, b1v), (sr0, sr1), (sa0, sa1)

    pltpu.sync_copy(eis.at[0, w2], idx)
    pltpu.async_copy(ms[0].at[pl.ds(w2 * ksl * CHUNK, CHUNK)], b0, sr0)

    pltpu.sync_copy(zrows.at[pl.ds(sid * rpt, rpt)],
                    shared.at[pl.ds(sid * rpt, rpt)])
    plsc.subcore_barrier()

    def wait_read(b):
        pltpu.make_async_copy(ms[0].at[pl.ds(0, CHUNK)], bufs[b], srs[b]).wait()

    def wait_add(b):
        pltpu.make_async_copy(ms[0].at[pl.ds(0, CHUNK)], bufs[b], sas[b]).wait()

    for s in range(len(ms)):
        def step(b, other):
            def body(j, carry):
                @pl.when(j >= 1)
                def _():
                    wait_add(other)

                @pl.when(j + 1 < ksl)
                def _():
                    row1 = (w2 * ksl + j + 1) * CHUNK
                    pltpu.async_copy(ms[s].at[pl.ds(row1, CHUNK)],
                                     bufs[other], srs[other])
                if s + 1 < len(ms):
                    @pl.when(j + 1 == ksl)
                    def _():
                        row1 = w2 * ksl * CHUNK
                        pltpu.async_copy(ms[s + 1].at[pl.ds(row1, CHUNK)],
                                         bufs[other], srs[other])
                wait_read(b)
                pltpu.async_copy(bufs[b], shared.at[idx.at[j]], sas[b],
                                 add=True)
                return carry
            return body

        def pairfn(p, carry):
            carry = step(0, 1)(2 * p, carry)
            carry = step(1, 0)(2 * p + 1, carry)
            return carry

        assert ksl % 2 == 0
        lax.fori_loop(0, ksl // 2, pairfn, 0)
        wait_add(1)
        if s + 1 < len(ms):
            pltpu.sync_copy(eis.at[s + 1, w2], idx)

    plsc.subcore_barrier()

    pltpu.sync_copy(shared.at[pl.ds(sid * rpt, rpt)],
                    out.at[pl.ds(cid * np_rows + sid * rpt, rpt)])


def _ceil_to(x, m):
    return (x + m - 1) // m * m


def kernel(h, e, d2, W1, b1, ln_g, ln_b, W2, b2, W3, b3, W4, b4):
    n, d = h.shape
    num_e = e.shape[1]
    ei = e[0]
    ej = e[1]

    ks = -(-num_e // (S * NS * CHUNK))
    eps = NS * ks * CHUNK
    ep = S * eps
    pad = ep - num_e
    ksl = eps // (NW * CHUNK)

    f32 = jnp.float32
    i32 = jnp.int32

    np_rows = _ceil_to(n + 1, NS * 8)

    h_p = jnp.pad(h, ((0, np_rows - n), (0, 0)))
    bn0 = np_rows // 16
    hA, hB = pl.pallas_call(
        _node_pre,
        grid=(16,),
        in_specs=[
            pl.BlockSpec((bn0, d), lambda i: (i, 0)),
            pl.BlockSpec((d, d), lambda i: (0, 0)),
            pl.BlockSpec((d, d), lambda i: (0, 0)),
            pl.BlockSpec((1, d), lambda i: (0, 0)),
        ],
        out_specs=[pl.BlockSpec((bn0, d), lambda i: (i, 0)),
                   pl.BlockSpec((bn0, d), lambda i: (i, 0))],
        out_shape=[jax.ShapeDtypeStruct((np_rows, d), f32)] * 2,
        compiler_params=pltpu.CompilerParams(
            dimension_semantics=("parallel",)),
    )(h_p, W1[:d], W1[d:2 * d], b1.reshape(1, d))

    ei4 = jnp.concatenate([ei, jnp.zeros((pad,), i32)]).reshape(S, NS, ks,
                                                                CHUNK)
    ej4 = jnp.concatenate([ej, jnp.zeros((pad,), i32)]).reshape(S, NS, ks,
                                                                CHUNK)
    d2p = jnp.concatenate([d2, jnp.zeros((pad, 1), f32)]).reshape(S, eps, 1)
    mesh = plsc.VectorSubcoreMesh(core_axis_name="c", subcore_axis_name="s",
                                  num_cores=NC, num_subcores=NS)
    gather = pl.kernel(
        _sc_gather,
        out_type=[jax.ShapeDtypeStruct((eps, d), f32)] * 2,
        mesh=mesh,
        scratch_types=[
            pltpu.VMEM((ks, CHUNK), i32),
            pltpu.VMEM((CHUNK, d), f32),
            pltpu.VMEM((CHUNK, d), f32),
            pltpu.SemaphoreType.DMA,
            pltpu.SemaphoreType.DMA,
            pltpu.SemaphoreType.DMA,
            pltpu.SemaphoreType.DMA,
            pltpu.VMEM_SHARED((np_rows, d), f32),
        ],
    )

    be = 2048
    mlp = pl.pallas_call(
        _edge_mlp,
        grid=(eps // be,),
        in_specs=[
            pl.BlockSpec((be, d), lambda i: (i, 0)),
            pl.BlockSpec((be, d), lambda i: (i, 0)),
            pl.BlockSpec((be, 1), lambda i: (i, 0)),
            pl.BlockSpec((1, d), lambda i: (0, 0)),
            pl.BlockSpec((1, d), lambda i: (0, 0)),
            pl.BlockSpec((1, d), lambda i: (0, 0)),
            pl.BlockSpec((d, d), lambda i: (0, 0)),
            pl.BlockSpec((1, d), lambda i: (0, 0)),
        ],
        out_specs=pl.BlockSpec((be, d), lambda i: (i, 0)),
        out_shape=jax.ShapeDtypeStruct((eps, d), f32),
        compiler_params=pltpu.CompilerParams(
            dimension_semantics=("parallel",)),
    )

    w1c = W1[2 * d].reshape(1, d)
    lngr = ln_g.reshape(1, d)
    lnbr = ln_b.reshape(1, d)
    b2r = b2.reshape(1, d)
    m_slices = []
    for s in range(S):
        gA, gB = gather(hA, hB, ei4[s], ej4[s])
        m_slices.append(mlp(gA, gB, d2p[s], w1c, lngr, lnbr, W2, b2r))

    ei_s = jnp.concatenate([ei, jnp.full((pad,), n, i32)]).reshape(S, NW, ksl,
                                                                   CHUNK)
    zrows = jnp.zeros((np_rows, d), f32)
    agg2 = pl.kernel(
        _sc_scatter,
        out_type=jax.ShapeDtypeStruct((NC * np_rows, d), f32),
        mesh=mesh,
        scratch_types=[
            pltpu.VMEM((ksl, CHUNK), i32),
            pltpu.VMEM((CHUNK, d), f32),
            pltpu.VMEM((CHUNK, d), f32),
            pltpu.SemaphoreType.DMA,
            pltpu.SemaphoreType.DMA,
            pltpu.SemaphoreType.DMA,
            pltpu.SemaphoreType.DMA,
            pltpu.VMEM_SHARED((np_rows, d), f32),
        ],
    )(*m_slices, ei_s, zrows)

    bn = 2000
    out = pl.pallas_call(
        _node_upd,
        grid=(n // bn,),
        in_specs=[
            pl.BlockSpec((bn, d), lambda i: (i, 0)),
            pl.BlockSpec((bn, d), lambda i: (i, 0)),
            pl.BlockSpec((bn, d), lambda i: (i, 0)),
            pl.BlockSpec((d, d), lambda i: (0, 0)),
            pl.BlockSpec((d, d), lambda i: (0, 0)),
            pl.BlockSpec((1, d), lambda i: (0, 0)),
            pl.BlockSpec((d, d), lambda i: (0, 0)),
            pl.BlockSpec((1, d), lambda i: (0, 0)),
        ],
        out_specs=pl.BlockSpec((bn, d), lambda i: (i, 0)),
        out_shape=jax.ShapeDtypeStruct((n, d), f32),
        compiler_params=pltpu.CompilerParams(
            dimension_semantics=("parallel",)),
    )(h, agg2[:n], agg2[np_rows:np_rows + n], W3[:d], W3[d:],
      b3.reshape(1, d), W4, b4.reshape(1, d))
    return out


# speedup vs baseline: 1.6626x; 1.0173x over previous
import jax
import jax.numpy as jnp
from jax import lax
from jax.experimental import pallas as pl
from jax.experimental.pallas import tpu as pltpu
from jax.experimental.pallas import tpu_sc as plsc

NC = 2
NS = 16
NW = NC * NS
CHUNK = 128
S = 4


def _node_pre(h_ref, wa_ref, wb_ref, b1_ref, ha_ref, hb_ref):
    h = h_ref[...]
    ha_ref[...] = jnp.dot(h, wa_ref[...], preferred_element_type=jnp.float32)
    hb_ref[...] = (jnp.dot(h, wb_ref[...], preferred_element_type=jnp.float32)
                   + b1_ref[...])


def _edge_mlp(ga_ref, gb_ref, d2_ref, w1c_ref, lng_ref, lnb_ref, w2_ref,
              b2_ref, m_ref):
    x = ga_ref[...] + gb_ref[...] + d2_ref[...] * w1c_ref[...]
    x = x * jax.nn.sigmoid(x)
    mu = jnp.mean(x, axis=-1, keepdims=True)
    var = jnp.mean((x - mu) ** 2, axis=-1, keepdims=True)
    x = (x - mu) * lax.rsqrt(var + 1e-5) * lng_ref[...] + lnb_ref[...]
    y = (jnp.dot(x.astype(jnp.bfloat16), w2_ref[...].astype(jnp.bfloat16),
                 preferred_element_type=jnp.float32) + b2_ref[...])
    m_ref[...] = y * jax.nn.sigmoid(y)


def _node_upd(h_ref, a0_ref, a1_ref, a2_ref, a3_ref, w3h_ref, w3a_ref, b3_ref,
              w4_ref, b4_ref, o_ref):
    h = h_ref[...]
    agg = (a0_ref[...] + a1_ref[...]) + (a2_ref[...] + a3_ref[...])
    y = (jnp.dot(h, w3h_ref[...], preferred_element_type=jnp.float32)
         + jnp.dot(agg, w3a_ref[...], preferred_element_type=jnp.float32)
         + b3_ref[...])
    y = y * jax.nn.sigmoid(y)
    o_ref[...] = h + jnp.dot(y, w4_ref[...], preferred_element_type=jnp.float32) + b4_ref[...]


def _sc_gather(ha, hb, ei3, ej3, ga, gb, idx, b0v, b1v, sg0, sg1, sw0, sw1,
               tab):
    ks = idx.shape[0]
    n2 = tab.shape[0]
    rpt = n2 // NS
    cid = lax.axis_index("c")
    sid = lax.axis_index("s")

    @pl.when(cid == 0)
    def _():
        pltpu.sync_copy(ha.at[pl.ds(sid * rpt, rpt)],
                        tab.at[pl.ds(sid * rpt, rpt)])
        pltpu.sync_copy(ei3.at[sid], idx)

    @pl.when(cid == 1)
    def _():
        pltpu.sync_copy(hb.at[pl.ds(sid * rpt, rpt)],
                        tab.at[pl.ds(sid * rpt, rpt)])
        pltpu.sync_copy(ej3.at[sid], idx)

    plsc.subcore_barrier()

    bufs, sgs, sws = (b0v, b1v), (sg0, sg1), (sw0, sw1)

    def wait(b, sems):
        pltpu.make_async_copy(ha.at[pl.ds(0, CHUNK)], bufs[b], sems[b]).wait()

    pltpu.async_copy(tab.at[idx.at[0]], bufs[0], sgs[0])

    def step(b, other):
        def body(j, carry):
            @pl.when(j + 1 < ks)
            def _():
                @pl.when(j >= 1)
                def _():
                    wait(other, sws)
                pltpu.async_copy(tab.at[idx.at[j + 1]], bufs[other],
                                 sgs[other])
            wait(b, sgs)
            row0 = (sid * ks + j) * CHUNK

            @pl.when(cid == 0)
            def _():
                pltpu.async_copy(bufs[b], ga.at[pl.ds(row0, CHUNK)], sws[b])

            @pl.when(cid == 1)
            def _():
                pltpu.async_copy(bufs[b], gb.at[pl.ds(row0, CHUNK)], sws[b])
            return carry
        return body

    def pairfn(p, carry):
        carry = step(0, 1)(2 * p, carry)
        carry = step(1, 0)(2 * p + 1, carry)
        return carry

    carry = lax.fori_loop(0, ks // 2, pairfn, 0)
    if ks % 2:
        step(0, 1)(ks - 1, carry)
    wait(0, sws)
    if ks > 1:
        wait(1, sws)


def _sc_scatter(m0, m1, eis, zrows, out, idx, b0, b1v,
                sr0, sr1, sa0, sa1, shared):
    ms = (m0, m1)
    ksl = idx.shape[0]
    np_rows = shared.shape[0]
    rpt = np_rows // NS
    cid = lax.axis_index("c")
    sid = lax.axis_index("s")
    w2 = cid * NS + sid
    bufs, srs, sas = (b0, b1v), (sr0, sr1), (sa0, sa1)

    pltpu.sync_copy(eis.at[0, w2], idx)
    pltpu.async_copy(ms[0].at[pl.ds(w2 * ksl * CHUNK, CHUNK)], b0, sr0)

    pltpu.sync_copy(zrows.at[pl.ds(sid * rpt, rpt)],
                    shared.at[pl.ds(sid * rpt, rpt)])
    plsc.subcore_barrier()

    def wait_read(b):
        pltpu.make_async_copy(ms[0].at[pl.ds(0, CHUNK)], bufs[b], srs[b]).wait()

    def wait_add(b):
        pltpu.make_async_copy(ms[0].at[pl.ds(0, CHUNK)], bufs[b], sas[b]).wait()

    for s in range(len(ms)):
        def step(b, other):
            def body(j, carry):
                @pl.when(j >= 1)
                def _():
                    wait_add(other)

                @pl.when(j + 1 < ksl)
                def _():
                    row1 = (w2 * ksl + j + 1) * CHUNK
                    pltpu.async_copy(ms[s].at[pl.ds(row1, CHUNK)],
                                     bufs[other], srs[other])
                if s + 1 < len(ms):
                    @pl.when(j + 1 == ksl)
                    def _():
                        row1 = w2 * ksl * CHUNK
                        pltpu.async_copy(ms[s + 1].at[pl.ds(row1, CHUNK)],
                                         bufs[other], srs[other])
                wait_read(b)
                pltpu.async_copy(bufs[b], shared.at[idx.at[j]], sas[b],
                                 add=True)
                return carry
            return body

        def pairfn(p, carry):
            carry = step(0, 1)(2 * p, carry)
            carry = step(1, 0)(2 * p + 1, carry)
            return carry

        assert ksl % 2 == 0
        lax.fori_loop(0, ksl // 2, pairfn, 0)
        wait_add(1)
        if s + 1 < len(ms):
            pltpu.sync_copy(eis.at[s + 1, w2], idx)

    plsc.subcore_barrier()

    pltpu.sync_copy(shared.at[pl.ds(sid * rpt, rpt)],
                    out.at[pl.ds(cid * np_rows + sid * rpt, rpt)])


def _ceil_to(x, m):
    return (x + m - 1) // m * m


def kernel(h, e, d2, W1, b1, ln_g, ln_b, W2, b2, W3, b3, W4, b4):
    n, d = h.shape
    num_e = e.shape[1]
    ei = e[0]
    ej = e[1]

    ks = -(-num_e // (S * NS * CHUNK))
    eps = NS * ks * CHUNK
    ep = S * eps
    pad = ep - num_e
    ksl = eps // (NW * CHUNK)

    f32 = jnp.float32
    i32 = jnp.int32

    np_rows = _ceil_to(n + 1, NS * 8)

    h_p = jnp.pad(h, ((0, np_rows - n), (0, 0)))
    bn0 = np_rows // 16
    hA, hB = pl.pallas_call(
        _node_pre,
        grid=(16,),
        in_specs=[
            pl.BlockSpec((bn0, d), lambda i: (i, 0)),
            pl.BlockSpec((d, d), lambda i: (0, 0)),
            pl.BlockSpec((d, d), lambda i: (0, 0)),
            pl.BlockSpec((1, d), lambda i: (0, 0)),
        ],
        out_specs=[pl.BlockSpec((bn0, d), lambda i: (i, 0)),
                   pl.BlockSpec((bn0, d), lambda i: (i, 0))],
        out_shape=[jax.ShapeDtypeStruct((np_rows, d), f32)] * 2,
        compiler_params=pltpu.CompilerParams(
            dimension_semantics=("parallel",)),
    )(h_p, W1[:d], W1[d:2 * d], b1.reshape(1, d))

    ei4 = jnp.concatenate([ei, jnp.zeros((pad,), i32)]).reshape(S, NS, ks,
                                                                CHUNK)
    ej4 = jnp.concatenate([ej, jnp.zeros((pad,), i32)]).reshape(S, NS, ks,
                                                                CHUNK)
    d2p = jnp.concatenate([d2, jnp.zeros((pad, 1), f32)]).reshape(S, eps, 1)
    mesh = plsc.VectorSubcoreMesh(core_axis_name="c", subcore_axis_name="s",
                                  num_cores=NC, num_subcores=NS)
    gather = pl.kernel(
        _sc_gather,
        out_type=[jax.ShapeDtypeStruct((eps, d), f32)] * 2,
        mesh=mesh,
        scratch_types=[
            pltpu.VMEM((ks, CHUNK), i32),
            pltpu.VMEM((CHUNK, d), f32),
            pltpu.VMEM((CHUNK, d), f32),
            pltpu.SemaphoreType.DMA,
            pltpu.SemaphoreType.DMA,
            pltpu.SemaphoreType.DMA,
            pltpu.SemaphoreType.DMA,
            pltpu.VMEM_SHARED((np_rows, d), f32),
        ],
    )

    be = 2048
    mlp = pl.pallas_call(
        _edge_mlp,
        grid=(eps // be,),
        in_specs=[
            pl.BlockSpec((be, d), lambda i: (i, 0)),
            pl.BlockSpec((be, d), lambda i: (i, 0)),
            pl.BlockSpec((be, 1), lambda i: (i, 0)),
            pl.BlockSpec((1, d), lambda i: (0, 0)),
            pl.BlockSpec((1, d), lambda i: (0, 0)),
            pl.BlockSpec((1, d), lambda i: (0, 0)),
            pl.BlockSpec((d, d), lambda i: (0, 0)),
            pl.BlockSpec((1, d), lambda i: (0, 0)),
        ],
        out_specs=pl.BlockSpec((be, d), lambda i: (i, 0)),
        out_shape=jax.ShapeDtypeStruct((eps, d), f32),
        compiler_params=pltpu.CompilerParams(
            dimension_semantics=("parallel",)),
    )

    w1c = W1[2 * d].reshape(1, d)
    lngr = ln_g.reshape(1, d)
    lnbr = ln_b.reshape(1, d)
    b2r = b2.reshape(1, d)
    m_slices = []
    for s in range(S):
        gA, gB = gather(hA, hB, ei4[s], ej4[s])
        m_slices.append(mlp(gA, gB, d2p[s], w1c, lngr, lnbr, W2, b2r))

    ei_s = jnp.concatenate([ei, jnp.full((pad,), n, i32)]).reshape(S, NW, ksl,
                                                                   CHUNK)
    zrows = jnp.zeros((np_rows, d), f32)
    scatter = pl.kernel(
        _sc_scatter,
        out_type=jax.ShapeDtypeStruct((NC * np_rows, d), f32),
        mesh=mesh,
        scratch_types=[
            pltpu.VMEM((ksl, CHUNK), i32),
            pltpu.VMEM((CHUNK, d), f32),
            pltpu.VMEM((CHUNK, d), f32),
            pltpu.SemaphoreType.DMA,
            pltpu.SemaphoreType.DMA,
            pltpu.SemaphoreType.DMA,
            pltpu.SemaphoreType.DMA,
            pltpu.VMEM_SHARED((np_rows, d), f32),
        ],
    )
    agg_g0 = scatter(m_slices[0], m_slices[1], ei_s[0:2], zrows)
    agg_g1 = scatter(m_slices[2], m_slices[3], ei_s[2:4], zrows)

    bn = 2000
    out = pl.pallas_call(
        _node_upd,
        grid=(n // bn,),
        in_specs=[
            pl.BlockSpec((bn, d), lambda i: (i, 0)),
            pl.BlockSpec((bn, d), lambda i: (i, 0)),
            pl.BlockSpec((bn, d), lambda i: (i, 0)),
            pl.BlockSpec((bn, d), lambda i: (i, 0)),
            pl.BlockSpec((bn, d), lambda i: (i, 0)),
            pl.BlockSpec((d, d), lambda i: (0, 0)),
            pl.BlockSpec((d, d), lambda i: (0, 0)),
            pl.BlockSpec((1, d), lambda i: (0, 0)),
            pl.BlockSpec((d, d), lambda i: (0, 0)),
            pl.BlockSpec((1, d), lambda i: (0, 0)),
        ],
        out_specs=pl.BlockSpec((bn, d), lambda i: (i, 0)),
        out_shape=jax.ShapeDtypeStruct((n, d), f32),
        compiler_params=pltpu.CompilerParams(
            dimension_semantics=("parallel",)),
    )(h, agg_g0[:n], agg_g0[np_rows:np_rows + n],
      agg_g1[:n], agg_g1[np_rows:np_rows + n], W3[:d], W3[d:],
      b3.reshape(1, d), W4, b4.reshape(1, d))
    return out
